# baseline (speedup 1.0000x reference)
import functools

import jax
import jax.numpy as jnp
from jax import lax
from jax.experimental import pallas as pl
from jax.experimental.pallas import tpu as pltpu
from jax.experimental.pallas import tpu_sc as plsc

NC = 2
NS = 16
NW = NC * NS
L = 16

DIM = 2048
HID = 2048
E = 16
TOPK = 2
T = 4096
NPAIR = T * TOPK
TILE = 256
NTILES = NPAIR // TILE + E
P = NTILES * TILE
GCHUNK = 1024
SCHUNK = 256


def _gate_kernel(x_ref, gw_ref, w_ref, i_ref, r_ref, off_ref, eid_ref,
                 carry_ref):
    step = pl.program_id(0)

    @pl.when(step == 0)
    def _():
        carry_ref[...] = jnp.zeros_like(carry_ref)

    @pl.when(step < T // TILE)
    def _():
        xb = x_ref[...]
        logits = jax.lax.dot_general(xb, gw_ref[...], (((1,), (1,)), ((), ())),
                                     preferred_element_type=jnp.float32)
        m = jnp.max(logits, axis=1, keepdims=True)
        p = jnp.exp(logits - m)
        s = p / jnp.sum(p, axis=1, keepdims=True)
        iota = jax.lax.broadcasted_iota(jnp.int32, s.shape, 1)
        m1 = jnp.max(s, axis=1, keepdims=True)
        i1 = jnp.min(jnp.where(s == m1, iota, E), axis=1, keepdims=True)
        s2 = jnp.where(iota == i1, -1.0, s)
        m2 = jnp.max(s2, axis=1, keepdims=True)
        i2 = jnp.min(jnp.where(s2 == m2, iota, E), axis=1, keepdims=True)
        w_ref[...] = jnp.concatenate([m1, m2], axis=1)
        i_ref[...] = jnp.concatenate([i1, i2], axis=1)
        oh0 = (i1 == iota).astype(jnp.float32)
        oh1 = (i2 == iota).astype(jnp.float32)
        ri = jax.lax.broadcasted_iota(jnp.int32, (TILE, TILE), 0)
        ci = jax.lax.broadcasted_iota(jnp.int32, (TILE, TILE), 1)
        tril = (ri > ci).astype(jnp.float32)
        carry = carry_ref[0:1, :]
        r0m = jax.lax.dot_general(tril, oh0, (((1,), (0,)), ((), ())),
                                  preferred_element_type=jnp.float32) + carry
        r0 = jnp.sum(r0m * oh0, axis=1, keepdims=True)
        carry = carry + jnp.sum(oh0, axis=0, keepdims=True)
        r1m = jax.lax.dot_general(tril, oh1, (((1,), (0,)), ((), ())),
                                  preferred_element_type=jnp.float32) + carry
        r1 = jnp.sum(r1m * oh1, axis=1, keepdims=True)
        carry = carry + jnp.sum(oh1, axis=0, keepdims=True)
        carry_ref[...] = jnp.broadcast_to(carry, carry_ref.shape)
        r_ref[...] = jnp.concatenate([r0, r1], axis=1).astype(jnp.int32)

    @pl.when(step == T // TILE)
    def _():
        counts = carry_ref[0:1, :].astype(jnp.int32)
        padded = ((counts + TILE - 1) // TILE) * TILE
        ri = jax.lax.broadcasted_iota(jnp.int32, (E, E), 0)
        ci = jax.lax.broadcasted_iota(jnp.int32, (E, E), 1)
        lm = (ri < ci).astype(jnp.float32)
        off = jax.lax.dot_general(padded.astype(jnp.float32), lm,
                                  (((1,), (0,)), ((), ())),
                                  preferred_element_type=jnp.float32)
        off = off.astype(jnp.int32)
        off_ref[...] = jnp.broadcast_to(off, off_ref.shape)
        ts = off // TILE
        eye = jnp.eye(E, dtype=jnp.float32)
        ts_sub = jax.lax.dot_general(eye, ts.astype(jnp.float32),
                                     (((1,), (1,)), ((), ())),
                                     preferred_element_type=jnp.float32)
        ti = jax.lax.broadcasted_iota(jnp.int32, (E, 64), 1).astype(
            jnp.float32)
        ge = (ti >= ts_sub).astype(jnp.float32)
        eid = jax.lax.dot_general(jnp.ones((1, E), jnp.float32), ge,
                                  (((1,), (0,)), ((), ())),
                                  preferred_element_type=jnp.float32) - 1.0
        eid = eid.astype(jnp.int32)
        iota_e = jax.lax.broadcasted_iota(jnp.int32, (1, E), 1)
        e_last = jnp.max(jnp.where(counts > 0, iota_e, 0), axis=1,
                         keepdims=True)
        eidm = jnp.minimum(eid, e_last)
        used = jnp.sum(padded, axis=1, keepdims=True) // TILE
        iota64 = jax.lax.broadcasted_iota(jnp.int32, (1, 64), 1)
        valid = (iota64 < used).astype(jnp.int32)
        rin = jnp.minimum(iota64, used - 1)
        rout = jnp.where(valid != 0, iota64, NTILES)
        meta = jnp.concatenate([eidm, rin, rout, valid], axis=1)
        eid_ref[...] = jnp.broadcast_to(meta, eid_ref.shape)


def _cast_kernel(x_ref, xb_ref):
    xb_ref[...] = x_ref[...].astype(jnp.bfloat16)


def _sharedh_kernel(x_ref, sw1_ref, sw2_ref, h_ref):
    xb = x_ref[...]
    s1b = sw1_ref[...].astype(jnp.bfloat16)
    s2b = sw2_ref[...].astype(jnp.bfloat16)
    a = jax.lax.dot_general(xb, s1b, (((1,), (1,)), ((), ())),
                            preferred_element_type=jnp.float32)
    b = jax.lax.dot_general(xb, s2b, (((1,), (1,)), ((), ())),
                            preferred_element_type=jnp.float32)
    h_ref[...] = (a * jax.nn.sigmoid(a) * b).astype(jnp.bfloat16)


def _sharedz_kernel(h_ref, sw3_ref, z_ref):
    k = pl.program_id(1)
    s3b = sw3_ref[...].astype(jnp.bfloat16)
    zc = jax.lax.dot_general(h_ref[...], s3b, (((1,), (1,)), ((), ())),
                             preferred_element_type=jnp.float32)

    @pl.when(k == 0)
    def _():
        z_ref[...] = zc

    @pl.when(k != 0)
    def _():
        z_ref[...] += zc


def _g1_kernel(eid_ref, rin_ref, rout_ref, vld_ref, xg_ref, ws_ref,
               w1_ref, w3_ref, h1_ref, h3_ref, w1b, w3b):
    del rin_ref, rout_ref
    i = pl.program_id(1)
    e = eid_ref[i]
    eprev = eid_ref[jnp.maximum(i, 1) - 1]

    @pl.when((i == 0) | (e != eprev))
    def _():
        w1b[...] = w1_ref[0].astype(jnp.bfloat16)
        w3b[...] = w3_ref[0].astype(jnp.bfloat16)

    @pl.when(vld_ref[i] != 0)
    def _():
        xi = (xg_ref[...].astype(jnp.float32)
              * ws_ref[...]).astype(jnp.bfloat16)
        a = jax.lax.dot_general(xi, w1b[...], (((1,), (1,)), ((), ())),
                                preferred_element_type=jnp.float32)
        h1_ref[...] = (a * jax.nn.sigmoid(a)).astype(jnp.bfloat16)
        h3_ref[...] = jax.lax.dot_general(
            xi, w3b[...], (((1,), (1,)), ((), ())),
            preferred_element_type=jnp.float32).astype(jnp.bfloat16)


def _g2_kernel(eid_ref, rin_ref, rout_ref, vld_ref, h1_ref, h3_ref, w2_ref,
               o_ref, w2b):
    del rin_ref, rout_ref
    i = pl.program_id(0)
    e = eid_ref[i]
    eprev = eid_ref[jnp.maximum(i, 1) - 1]

    @pl.when((i == 0) | (e != eprev))
    def _():
        w2b[...] = w2_ref[0].astype(jnp.bfloat16)

    @pl.when(vld_ref[i] != 0)
    def _():
        o_ref[...] = jax.lax.dot_general(
            h1_ref[...], w2b[...], (((1,), (1,)), ((), ())),
            preferred_element_type=jnp.float32) \
            * h3_ref[...].astype(jnp.float32)


def _worker_id():
    return lax.axis_index("s") * NC + lax.axis_index("c")


def _dispatch_sc(idx_hbm, wts_hbm, rank_hbm, off_hbm, tok_hbm, ws_hbm,
                 pos_hbm, idx_v, wts_v, rank_v, off_v, tok_v, ws_v, pos_v):
    @pl.when(_worker_id() == 0)
    def _():
        pltpu.sync_copy(idx_hbm, idx_v)
        pltpu.sync_copy(wts_hbm, wts_v)
        pltpu.sync_copy(rank_hbm, rank_v)
        pltpu.sync_copy(off_hbm.at[0], off_v)

        def zbody(i, _):
            sl = pl.ds(i * L, L)
            tok_v[sl] = jnp.zeros((L,), jnp.int32)
            ws_v[sl] = jnp.zeros((L,), jnp.float32)
            return ()
        lax.fori_loop(0, P // L, zbody, ())

        iota = lax.broadcasted_iota(jnp.int32, (L,), 0)

        def body(i, _):
            sl = pl.ds(i * L, L)
            e16 = idx_v[sl]
            o16 = plsc.load_gather(off_v, [e16])
            p16 = rank_v[sl] + o16
            pos_v[sl] = p16
            t16 = lax.shift_right_logical(iota + i * L, 1)
            plsc.store_scatter(tok_v, [p16], t16)
            plsc.store_scatter(ws_v, [p16], wts_v[sl])
            return ()
        lax.fori_loop(0, NPAIR // L, body, ())
        pltpu.sync_copy(tok_v, tok_hbm)
        pltpu.sync_copy(ws_v, ws_hbm)
        pltpu.sync_copy(pos_v, pos_hbm)


GCH = 24


def _gather_sc(x_hbm, tok_hbm, xg_hbm, idx_v, row_a, row_b,
               sga, sgb, ssa, ssb):
    rows_w = P // NW
    nch = rows_w // GCH
    base = _worker_id() * rows_w
    pltpu.sync_copy(tok_hbm.at[pl.ds(base, rows_w)], idx_v)

    def g_start(c, buf, sem):
        pltpu.make_async_copy(x_hbm.at[idx_v.at[pl.ds(c * GCH, GCH)]],
                              buf, sem).start()

    def s_start(c, buf, sem):
        pltpu.make_async_copy(buf, xg_hbm.at[pl.ds(base + c * GCH, GCH)],
                              sem).start()

    def g_wait(buf, sem):
        pltpu.make_async_copy(x_hbm.at[idx_v.at[pl.ds(0, GCH)]],
                              buf, sem).wait()

    def s_wait(buf, sem):
        pltpu.make_async_copy(buf, xg_hbm.at[pl.ds(base, GCH)], sem).wait()

    g_start(0, row_a, sga)

    def body(i, _):
        @pl.when(i > 0)
        def _():
            s_wait(row_b, ssb)
        g_start(2 * i + 1, row_b, sgb)
        g_wait(row_a, sga)
        s_start(2 * i, row_a, ssa)

        @pl.when(i < nch // 2 - 1)
        def _():
            s_wait(row_a, ssa)
            g_start(2 * i + 2, row_a, sga)
        g_wait(row_b, sgb)
        s_start(2 * i + 1, row_b, ssb)
        return ()
    lax.fori_loop(0, nch // 2, body, ())
    s_wait(row_a, ssa)
    s_wait(row_b, ssb)


CB = 4


def _combine_sc(og_hbm, z_hbm, pos_hbm, y_hbm, pos_v, row_a, row_b,
                z_a, z_b, y_a, y_b, sga, sgb, sza, szb, sya, syb):
    tok_w = T // NW
    nsb = tok_w // CB
    tb = _worker_id() * tok_w
    pltpu.sync_copy(pos_hbm.at[pl.ds(tb * 2, tok_w * 2)], pos_v)

    def fetch_start(s, rows, zv, sg, sz):
        t0 = tb + s * CB
        pltpu.make_async_copy(og_hbm.at[pos_v.at[pl.ds(s * 2 * CB, 2 * CB)]],
                              rows, sg).start()
        pltpu.make_async_copy(z_hbm.at[pl.ds(t0, CB)], zv, sz).start()

    def fetch_wait(rows, zv, sg, sz):
        pltpu.make_async_copy(og_hbm.at[pos_v.at[pl.ds(0, 2 * CB)]],
                              rows, sg).wait()
        pltpu.make_async_copy(z_hbm.at[pl.ds(tb, CB)], zv, sz).wait()

    def compute_store(s, rows, zv, yv, sy):
        def cbody(c, _):
            sl = pl.ds(c * L, L)
            for k in range(CB):
                yv[k, sl] = rows[2 * k, sl] + rows[2 * k + 1, sl] + zv[k, sl]
            return ()
        lax.fori_loop(0, DIM // L, cbody, ())
        pltpu.make_async_copy(yv, y_hbm.at[pl.ds(tb + s * CB, CB)],
                              sy).start()

    def y_wait(yv, sy):
        pltpu.make_async_copy(yv, y_hbm.at[pl.ds(tb, CB)], sy).wait()

    fetch_start(0, row_a, z_a, sga, sza)

    def body(i, _):
        @pl.when(i > 0)
        def _():
            y_wait(y_b, syb)
        fetch_start(2 * i + 1, row_b, z_b, sgb, szb)
        fetch_wait(row_a, z_a, sga, sza)
        compute_store(2 * i, row_a, z_a, y_a, sya)

        @pl.when(i < nsb // 2 - 1)
        def _():
            y_wait(y_a, sya)
            fetch_start(2 * i + 2, row_a, z_a, sga, sza)
        fetch_wait(row_b, z_b, sgb, szb)
        compute_store(2 * i + 1, row_b, z_b, y_b, syb)
        return ()
    lax.fori_loop(0, nsb // 2, body, ())
    y_wait(y_a, sya)
    y_wait(y_b, syb)


def kernel(x, gate_w, w1, b1, w2, b2, w3, b3, sw1, sw2, sw3):
    del b1, b2, b3
    nblk = T // TILE
    weights, indices, rank, off_m, eid_m = pl.pallas_call(
        _gate_kernel,
        grid=(nblk + 1,),
        in_specs=[pl.BlockSpec((TILE, DIM),
                               lambda i: (jnp.minimum(i, nblk - 1), 0)),
                  pl.BlockSpec((E, DIM), lambda i: (0, 0))],
        out_specs=[pl.BlockSpec((TILE, TOPK),
                                lambda i: (jnp.minimum(i, nblk - 1), 0)),
                   pl.BlockSpec((TILE, TOPK),
                                lambda i: (jnp.minimum(i, nblk - 1), 0)),
                   pl.BlockSpec((TILE, TOPK),
                                lambda i: (jnp.minimum(i, nblk - 1), 0)),
                   pl.BlockSpec((8, E), lambda i: (0, 0)),
                   pl.BlockSpec((8, 256), lambda i: (0, 0))],
        out_shape=[jax.ShapeDtypeStruct((T, TOPK), jnp.float32),
                   jax.ShapeDtypeStruct((T, TOPK), jnp.int32),
                   jax.ShapeDtypeStruct((T, TOPK), jnp.int32),
                   jax.ShapeDtypeStruct((8, E), jnp.int32),
                   jax.ShapeDtypeStruct((8, 256), jnp.int32)],
        scratch_shapes=[pltpu.VMEM((8, E), jnp.float32)],
    )(x, gate_w)

    eid = eid_m[0, :NTILES]
    rin = eid_m[0, 64:64 + NTILES]
    rout = eid_m[0, 128:128 + NTILES]
    vld = eid_m[0, 192:192 + NTILES]
    mesh = plsc.VectorSubcoreMesh(core_axis_name="c", subcore_axis_name="s",
                                  num_cores=NC, num_subcores=NS)
    tok_sorted, ws, pos = pl.kernel(
        _dispatch_sc,
        out_type=[jax.ShapeDtypeStruct((P,), jnp.int32),
                  jax.ShapeDtypeStruct((P,), jnp.float32),
                  jax.ShapeDtypeStruct((NPAIR,), jnp.int32)],
        mesh=mesh,
        scratch_types=[pltpu.VMEM((NPAIR,), jnp.int32),
                       pltpu.VMEM((NPAIR,), jnp.float32),
                       pltpu.VMEM((NPAIR,), jnp.int32),
                       pltpu.VMEM((E,), jnp.int32),
                       pltpu.VMEM((P,), jnp.int32),
                       pltpu.VMEM((P,), jnp.float32),
                       pltpu.VMEM((NPAIR,), jnp.int32)],
        compiler_params=pltpu.CompilerParams(needs_layout_passes=False),
    )(indices.reshape(-1), weights.reshape(-1), rank.reshape(-1), off_m)

    xb16 = pl.pallas_call(
        _cast_kernel,
        grid=(T // TILE,),
        in_specs=[pl.BlockSpec((TILE, DIM), lambda i: (i, 0))],
        out_specs=pl.BlockSpec((TILE, DIM), lambda i: (i, 0)),
        out_shape=jax.ShapeDtypeStruct((T, DIM), jnp.bfloat16),
    )(x)

    x2 = lax.bitcast_convert_type(xb16.reshape(T, DIM // 2, 2), jnp.int32)
    xgi = pl.kernel(
        _gather_sc,
        out_type=jax.ShapeDtypeStruct((P, DIM // 2), jnp.int32),
        mesh=mesh,
        scratch_types=[pltpu.VMEM((P // NW,), jnp.int32),
                       pltpu.VMEM((GCH, DIM // 2), jnp.int32),
                       pltpu.VMEM((GCH, DIM // 2), jnp.int32),
                       pltpu.SemaphoreType.DMA,
                       pltpu.SemaphoreType.DMA,
                       pltpu.SemaphoreType.DMA,
                       pltpu.SemaphoreType.DMA],
    )(x2, tok_sorted)
    xg = lax.bitcast_convert_type(xgi, jnp.bfloat16).reshape(P, DIM)

    hsh = pl.pallas_call(
        _sharedh_kernel,
        grid=(2 * HID // SCHUNK,),
        in_specs=[pl.BlockSpec((T, DIM), lambda c: (0, 0)),
                  pl.BlockSpec((SCHUNK, DIM), lambda c: (c, 0)),
                  pl.BlockSpec((SCHUNK, DIM), lambda c: (c, 0))],
        out_specs=pl.BlockSpec((T, SCHUNK), lambda c: (0, c)),
        out_shape=jax.ShapeDtypeStruct((T, 2 * HID), jnp.bfloat16),
        compiler_params=pltpu.CompilerParams(
            dimension_semantics=("arbitrary",)),
    )(xb16, sw1, sw2)

    ZT, ZK = 1024, 1024
    z = pl.pallas_call(
        _sharedz_kernel,
        grid=(T // ZT, 2 * HID // ZK),
        in_specs=[pl.BlockSpec((ZT, ZK), lambda t, k: (t, k)),
                  pl.BlockSpec((DIM, ZK), lambda t, k: (0, k))],
        out_specs=pl.BlockSpec((ZT, DIM), lambda t, k: (t, 0)),
        out_shape=jax.ShapeDtypeStruct((T, DIM), jnp.float32),
        compiler_params=pltpu.CompilerParams(
            dimension_semantics=("arbitrary", "arbitrary")),
    )(hsh, sw3)

    h1, h3 = pl.pallas_call(
        _g1_kernel,
        grid_spec=pltpu.PrefetchScalarGridSpec(
            num_scalar_prefetch=4,
            grid=(HID // GCHUNK, NTILES),
            in_specs=[
                pl.BlockSpec((TILE, DIM), lambda j, i, e, r, o, v: (r[i], 0)),
                pl.BlockSpec((TILE, 1), lambda j, i, e, r, o, v: (r[i], 0)),
                pl.BlockSpec((1, GCHUNK, DIM),
                             lambda j, i, e, r, o, v: (e[i], j, 0)),
                pl.BlockSpec((1, GCHUNK, DIM),
                             lambda j, i, e, r, o, v: (e[i], j, 0)),
            ],
            out_specs=[
                pl.BlockSpec((TILE, GCHUNK),
                             lambda j, i, e, r, o, v: (o[i], j)),
                pl.BlockSpec((TILE, GCHUNK),
                             lambda j, i, e, r, o, v: (o[i], j)),
            ],
            scratch_shapes=[pltpu.VMEM((GCHUNK, DIM), jnp.bfloat16),
                            pltpu.VMEM((GCHUNK, DIM), jnp.bfloat16)],
        ),
        out_shape=[jax.ShapeDtypeStruct((P + TILE, HID), jnp.bfloat16),
                   jax.ShapeDtypeStruct((P + TILE, HID), jnp.bfloat16)],
        compiler_params=pltpu.CompilerParams(
            dimension_semantics=("arbitrary", "arbitrary")),
    )(eid, rin, rout, vld, xg, ws.reshape(P, 1), w1, w3)

    og = pl.pallas_call(
        _g2_kernel,
        grid_spec=pltpu.PrefetchScalarGridSpec(
            num_scalar_prefetch=4,
            grid=(NTILES,),
            in_specs=[
                pl.BlockSpec((TILE, HID), lambda i, e, r, o, v: (r[i], 0)),
                pl.BlockSpec((TILE, HID), lambda i, e, r, o, v: (r[i], 0)),
                pl.BlockSpec((1, DIM, HID),
                             lambda i, e, r, o, v: (e[i], 0, 0)),
            ],
            out_specs=pl.BlockSpec((TILE, DIM),
                                   lambda i, e, r, o, v: (o[i], 0)),
            scratch_shapes=[pltpu.VMEM((DIM, HID), jnp.bfloat16)],
        ),
        out_shape=jax.ShapeDtypeStruct((P + TILE, DIM), jnp.float32),
        compiler_params=pltpu.CompilerParams(
            dimension_semantics=("arbitrary",)),
    )(eid, rin, rout, vld, h1, h3, w2)

    y = pl.kernel(
        _combine_sc,
        out_type=jax.ShapeDtypeStruct((T, DIM), jnp.float32),
        mesh=mesh,
        scratch_types=[pltpu.VMEM((2 * T // NW,), jnp.int32),
                       pltpu.VMEM((2 * CB, DIM), jnp.float32),
                       pltpu.VMEM((2 * CB, DIM), jnp.float32),
                       pltpu.VMEM((CB, DIM), jnp.float32),
                       pltpu.VMEM((CB, DIM), jnp.float32),
                       pltpu.VMEM((CB, DIM), jnp.float32),
                       pltpu.VMEM((CB, DIM), jnp.float32),
                       pltpu.SemaphoreType.DMA,
                       pltpu.SemaphoreType.DMA,
                       pltpu.SemaphoreType.DMA,
                       pltpu.SemaphoreType.DMA,
                       pltpu.SemaphoreType.DMA,
                       pltpu.SemaphoreType.DMA],
    )(og, z, pos)
    return y


# speedup vs baseline: 1.3601x; 1.3601x over previous
import functools

import jax
import jax.numpy as jnp
from jax import lax
from jax.experimental import pallas as pl
from jax.experimental.pallas import tpu as pltpu
from jax.experimental.pallas import tpu_sc as plsc

NC = 2
NS = 16
NW = NC * NS
L = 16

DIM = 2048
HID = 2048
E = 16
TOPK = 2
T = 4096
NPAIR = T * TOPK
TILE = 256
NTILES = NPAIR // TILE + E
P = NTILES * TILE
GCHUNK = 512
SCHUNK = 256


def _gate_kernel(x_ref, gw_ref, w_ref, i_ref, r_ref, off_ref, eid_ref,
                 carry_ref):
    step = pl.program_id(0)

    @pl.when(step == 0)
    def _():
        carry_ref[...] = jnp.zeros_like(carry_ref)

    @pl.when(step < T // TILE)
    def _():
        xb = x_ref[...]
        logits = jax.lax.dot_general(xb, gw_ref[...], (((1,), (1,)), ((), ())),
                                     preferred_element_type=jnp.float32)
        m = jnp.max(logits, axis=1, keepdims=True)
        p = jnp.exp(logits - m)
        s = p / jnp.sum(p, axis=1, keepdims=True)
        iota = jax.lax.broadcasted_iota(jnp.int32, s.shape, 1)
        m1 = jnp.max(s, axis=1, keepdims=True)
        i1 = jnp.min(jnp.where(s == m1, iota, E), axis=1, keepdims=True)
        s2 = jnp.where(iota == i1, -1.0, s)
        m2 = jnp.max(s2, axis=1, keepdims=True)
        i2 = jnp.min(jnp.where(s2 == m2, iota, E), axis=1, keepdims=True)
        w_ref[...] = jnp.concatenate([m1, m2], axis=1)
        i_ref[...] = jnp.concatenate([i1, i2], axis=1)
        oh0 = (i1 == iota).astype(jnp.float32)
        oh1 = (i2 == iota).astype(jnp.float32)
        ri = jax.lax.broadcasted_iota(jnp.int32, (TILE, TILE), 0)
        ci = jax.lax.broadcasted_iota(jnp.int32, (TILE, TILE), 1)
        tril = (ri > ci).astype(jnp.float32)
        carry = carry_ref[0:1, :]
        r0m = jax.lax.dot_general(tril, oh0, (((1,), (0,)), ((), ())),
                                  preferred_element_type=jnp.float32) + carry
        r0 = jnp.sum(r0m * oh0, axis=1, keepdims=True)
        carry = carry + jnp.sum(oh0, axis=0, keepdims=True)
        r1m = jax.lax.dot_general(tril, oh1, (((1,), (0,)), ((), ())),
                                  preferred_element_type=jnp.float32) + carry
        r1 = jnp.sum(r1m * oh1, axis=1, keepdims=True)
        carry = carry + jnp.sum(oh1, axis=0, keepdims=True)
        carry_ref[...] = jnp.broadcast_to(carry, carry_ref.shape)
        r_ref[...] = jnp.concatenate([r0, r1], axis=1).astype(jnp.int32)

    @pl.when(step == T // TILE)
    def _():
        counts = carry_ref[0:1, :].astype(jnp.int32)
        padded = ((counts + TILE - 1) // TILE) * TILE
        ri = jax.lax.broadcasted_iota(jnp.int32, (E, E), 0)
        ci = jax.lax.broadcasted_iota(jnp.int32, (E, E), 1)
        lm = (ri < ci).astype(jnp.float32)
        off = jax.lax.dot_general(padded.astype(jnp.float32), lm,
                                  (((1,), (0,)), ((), ())),
                                  preferred_element_type=jnp.float32)
        off = off.astype(jnp.int32)
        off_ref[...] = jnp.broadcast_to(off, off_ref.shape)
        ts = off // TILE
        eye = jnp.eye(E, dtype=jnp.float32)
        ts_sub = jax.lax.dot_general(eye, ts.astype(jnp.float32),
                                     (((1,), (1,)), ((), ())),
                                     preferred_element_type=jnp.float32)
        ti = jax.lax.broadcasted_iota(jnp.int32, (E, 64), 1).astype(
            jnp.float32)
        ge = (ti >= ts_sub).astype(jnp.float32)
        eid = jax.lax.dot_general(jnp.ones((1, E), jnp.float32), ge,
                                  (((1,), (0,)), ((), ())),
                                  preferred_element_type=jnp.float32) - 1.0
        eid = eid.astype(jnp.int32)
        iota_e = jax.lax.broadcasted_iota(jnp.int32, (1, E), 1)
        e_last = jnp.max(jnp.where(counts > 0, iota_e, 0), axis=1,
                         keepdims=True)
        eidm = jnp.minimum(eid, e_last)
        used = jnp.sum(padded, axis=1, keepdims=True) // TILE
        iota64 = jax.lax.broadcasted_iota(jnp.int32, (1, 64), 1)
        valid = (iota64 < used).astype(jnp.int32)
        rin = jnp.minimum(iota64, used - 1)
        rout = jnp.where(valid != 0, iota64, NTILES)
        meta = jnp.concatenate([eidm, rin, rout, valid], axis=1)
        eid_ref[...] = jnp.broadcast_to(meta, eid_ref.shape)


def _cast_kernel(x_ref, xb_ref):
    xb_ref[...] = x_ref[...].astype(jnp.bfloat16)


def _sharedh_kernel(x_ref, sw1_ref, sw2_ref, h_ref):
    xb = x_ref[...]
    s1b = sw1_ref[...].astype(jnp.bfloat16)
    s2b = sw2_ref[...].astype(jnp.bfloat16)
    a = jax.lax.dot_general(xb, s1b, (((1,), (1,)), ((), ())),
                            preferred_element_type=jnp.float32)
    b = jax.lax.dot_general(xb, s2b, (((1,), (1,)), ((), ())),
                            preferred_element_type=jnp.float32)
    h_ref[...] = (a * jax.nn.sigmoid(a) * b).astype(jnp.bfloat16)


def _sharedz_kernel(h_ref, sw3_ref, z_ref):
    k = pl.program_id(1)
    s3b = sw3_ref[...].astype(jnp.bfloat16)
    zc = jax.lax.dot_general(h_ref[...], s3b, (((1,), (1,)), ((), ())),
                             preferred_element_type=jnp.float32)

    @pl.when(k == 0)
    def _():
        z_ref[...] = zc

    @pl.when(k != 0)
    def _():
        z_ref[...] += zc


def _g_kernel(eid_ref, rin_ref, rout_ref, vld_ref, xg_ref, ws_ref,
              w1_ref, w2_ref, w3_ref, o_ref, h3s):
    del eid_ref, rin_ref, rout_ref
    i = pl.program_id(0)
    j = pl.program_id(1)
    nj = pl.num_programs(1)

    @pl.when(vld_ref[i] != 0)
    def _():
        xi = (xg_ref[...] * ws_ref[...]).astype(jnp.bfloat16)
        w1b = w1_ref[0].astype(jnp.bfloat16)
        a = jax.lax.dot_general(xi, w1b, (((1,), (1,)), ((), ())),
                                preferred_element_type=jnp.float32)
        h1j = (a * jax.nn.sigmoid(a)).astype(jnp.bfloat16)
        w2b = w2_ref[0].astype(jnp.bfloat16)
        part = jax.lax.dot_general(h1j, w2b, (((1,), (1,)), ((), ())),
                                   preferred_element_type=jnp.float32)
        w3b = w3_ref[0].astype(jnp.bfloat16)
        h3s[:, pl.ds(j * GCHUNK, GCHUNK)] = jax.lax.dot_general(
            xi, w3b, (((1,), (1,)), ((), ())),
            preferred_element_type=jnp.float32)

        @pl.when(j == 0)
        def _():
            o_ref[...] = part

        @pl.when(j != 0)
        def _():
            o_ref[...] += part

        @pl.when(j == nj - 1)
        def _():
            o_ref[...] *= h3s[...]


def _worker_id():
    return lax.axis_index("s") * NC + lax.axis_index("c")


def _dispatch_sc(idx_hbm, wts_hbm, rank_hbm, off_hbm, tok_hbm, ws_hbm,
                 pos_hbm, idx_v, wts_v, rank_v, off_v, tok_v, ws_v, pos_v):
    @pl.when(_worker_id() == 0)
    def _():
        pltpu.sync_copy(idx_hbm, idx_v)
        pltpu.sync_copy(wts_hbm, wts_v)
        pltpu.sync_copy(rank_hbm, rank_v)
        pltpu.sync_copy(off_hbm.at[0], off_v)

        def zbody(i, _):
            sl = pl.ds(i * L, L)
            tok_v[sl] = jnp.zeros((L,), jnp.int32)
            ws_v[sl] = jnp.zeros((L,), jnp.float32)
            return ()
        lax.fori_loop(0, P // L, zbody, ())

        iota = lax.broadcasted_iota(jnp.int32, (L,), 0)

        def body(i, _):
            sl = pl.ds(i * L, L)
            e16 = idx_v[sl]
            o16 = plsc.load_gather(off_v, [e16])
            p16 = rank_v[sl] + o16
            pos_v[sl] = p16
            t16 = lax.shift_right_logical(iota + i * L, 1)
            plsc.store_scatter(tok_v, [p16], t16)
            plsc.store_scatter(ws_v, [p16], wts_v[sl])
            return ()
        lax.fori_loop(0, NPAIR // L, body, ())
        pltpu.sync_copy(tok_v, tok_hbm)
        pltpu.sync_copy(ws_v, ws_hbm)
        pltpu.sync_copy(pos_v, pos_hbm)


GCH = 24


def _gather_sc(x_hbm, tok_hbm, xg_hbm, idx_v, row_a, row_b,
               sga, sgb, ssa, ssb):
    rows_w = P // NW
    nch = rows_w // GCH
    base = _worker_id() * rows_w
    pltpu.sync_copy(tok_hbm.at[pl.ds(base, rows_w)], idx_v)

    def g_start(c, buf, sem):
        pltpu.make_async_copy(x_hbm.at[idx_v.at[pl.ds(c * GCH, GCH)]],
                              buf, sem).start()

    def s_start(c, buf, sem):
        pltpu.make_async_copy(buf, xg_hbm.at[pl.ds(base + c * GCH, GCH)],
                              sem).start()

    def g_wait(buf, sem):
        pltpu.make_async_copy(x_hbm.at[idx_v.at[pl.ds(0, GCH)]],
                              buf, sem).wait()

    def s_wait(buf, sem):
        pltpu.make_async_copy(buf, xg_hbm.at[pl.ds(base, GCH)], sem).wait()

    g_start(0, row_a, sga)

    def body(i, _):
        @pl.when(i > 0)
        def _():
            s_wait(row_b, ssb)
        g_start(2 * i + 1, row_b, sgb)
        g_wait(row_a, sga)
        s_start(2 * i, row_a, ssa)

        @pl.when(i < nch // 2 - 1)
        def _():
            s_wait(row_a, ssa)
            g_start(2 * i + 2, row_a, sga)
        g_wait(row_b, sgb)
        s_start(2 * i + 1, row_b, ssb)
        return ()
    lax.fori_loop(0, nch // 2, body, ())
    s_wait(row_a, ssa)
    s_wait(row_b, ssb)


CB = 4


def _combine_sc(og_hbm, z_hbm, pos_hbm, y_hbm, pos_v, row_a, row_b,
                z_a, z_b, y_a, y_b, sga, sgb, sza, szb, sya, syb):
    tok_w = T // NW
    nsb = tok_w // CB
    tb = _worker_id() * tok_w
    pltpu.sync_copy(pos_hbm.at[pl.ds(tb * 2, tok_w * 2)], pos_v)

    def fetch_start(s, rows, zv, sg, sz):
        t0 = tb + s * CB
        pltpu.make_async_copy(og_hbm.at[pos_v.at[pl.ds(s * 2 * CB, 2 * CB)]],
                              rows, sg).start()
        pltpu.make_async_copy(z_hbm.at[pl.ds(t0, CB)], zv, sz).start()

    def fetch_wait(rows, zv, sg, sz):
        pltpu.make_async_copy(og_hbm.at[pos_v.at[pl.ds(0, 2 * CB)]],
                              rows, sg).wait()
        pltpu.make_async_copy(z_hbm.at[pl.ds(tb, CB)], zv, sz).wait()

    def compute_store(s, rows, zv, yv, sy):
        def cbody(c, _):
            sl = pl.ds(c * L, L)
            for k in range(CB):
                yv[k, sl] = rows[2 * k, sl] + rows[2 * k + 1, sl] + zv[k, sl]
            return ()
        lax.fori_loop(0, DIM // L, cbody, ())
        pltpu.make_async_copy(yv, y_hbm.at[pl.ds(tb + s * CB, CB)],
                              sy).start()

    def y_wait(yv, sy):
        pltpu.make_async_copy(yv, y_hbm.at[pl.ds(tb, CB)], sy).wait()

    fetch_start(0, row_a, z_a, sga, sza)

    def body(i, _):
        @pl.when(i > 0)
        def _():
            y_wait(y_b, syb)
        fetch_start(2 * i + 1, row_b, z_b, sgb, szb)
        fetch_wait(row_a, z_a, sga, sza)
        compute_store(2 * i, row_a, z_a, y_a, sya)

        @pl.when(i < nsb // 2 - 1)
        def _():
            y_wait(y_a, sya)
            fetch_start(2 * i + 2, row_a, z_a, sga, sza)
        fetch_wait(row_b, z_b, sgb, szb)
        compute_store(2 * i + 1, row_b, z_b, y_b, syb)
        return ()
    lax.fori_loop(0, nsb // 2, body, ())
    y_wait(y_a, sya)
    y_wait(y_b, syb)


def kernel(x, gate_w, w1, b1, w2, b2, w3, b3, sw1, sw2, sw3):
    del b1, b2, b3
    nblk = T // TILE
    weights, indices, rank, off_m, eid_m = pl.pallas_call(
        _gate_kernel,
        grid=(nblk + 1,),
        in_specs=[pl.BlockSpec((TILE, DIM),
                               lambda i: (jnp.minimum(i, nblk - 1), 0)),
                  pl.BlockSpec((E, DIM), lambda i: (0, 0))],
        out_specs=[pl.BlockSpec((TILE, TOPK),
                                lambda i: (jnp.minimum(i, nblk - 1), 0)),
                   pl.BlockSpec((TILE, TOPK),
                                lambda i: (jnp.minimum(i, nblk - 1), 0)),
                   pl.BlockSpec((TILE, TOPK),
                                lambda i: (jnp.minimum(i, nblk - 1), 0)),
                   pl.BlockSpec((8, E), lambda i: (0, 0)),
                   pl.BlockSpec((8, 256), lambda i: (0, 0))],
        out_shape=[jax.ShapeDtypeStruct((T, TOPK), jnp.float32),
                   jax.ShapeDtypeStruct((T, TOPK), jnp.int32),
                   jax.ShapeDtypeStruct((T, TOPK), jnp.int32),
                   jax.ShapeDtypeStruct((8, E), jnp.int32),
                   jax.ShapeDtypeStruct((8, 256), jnp.int32)],
        scratch_shapes=[pltpu.VMEM((8, E), jnp.float32)],
    )(x, gate_w)

    eid = eid_m[0, :NTILES]
    rin = eid_m[0, 64:64 + NTILES]
    rout = eid_m[0, 128:128 + NTILES]
    vld = eid_m[0, 192:192 + NTILES]
    mesh = plsc.VectorSubcoreMesh(core_axis_name="c", subcore_axis_name="s",
                                  num_cores=NC, num_subcores=NS)
    tok_sorted, ws, pos = pl.kernel(
        _dispatch_sc,
        out_type=[jax.ShapeDtypeStruct((P,), jnp.int32),
                  jax.ShapeDtypeStruct((P,), jnp.float32),
                  jax.ShapeDtypeStruct((NPAIR,), jnp.int32)],
        mesh=mesh,
        scratch_types=[pltpu.VMEM((NPAIR,), jnp.int32),
                       pltpu.VMEM((NPAIR,), jnp.float32),
                       pltpu.VMEM((NPAIR,), jnp.int32),
                       pltpu.VMEM((E,), jnp.int32),
                       pltpu.VMEM((P,), jnp.int32),
                       pltpu.VMEM((P,), jnp.float32),
                       pltpu.VMEM((NPAIR,), jnp.int32)],
        compiler_params=pltpu.CompilerParams(needs_layout_passes=False),
    )(indices.reshape(-1), weights.reshape(-1), rank.reshape(-1), off_m)

    xb16 = pl.pallas_call(
        _cast_kernel,
        grid=(T // TILE,),
        in_specs=[pl.BlockSpec((TILE, DIM), lambda i: (i, 0))],
        out_specs=pl.BlockSpec((TILE, DIM), lambda i: (i, 0)),
        out_shape=jax.ShapeDtypeStruct((T, DIM), jnp.bfloat16),
    )(x)

    xg = pl.kernel(
        _gather_sc,
        out_type=jax.ShapeDtypeStruct((P, DIM), jnp.float32),
        mesh=mesh,
        scratch_types=[pltpu.VMEM((P // NW,), jnp.int32),
                       pltpu.VMEM((GCH, DIM), jnp.float32),
                       pltpu.VMEM((GCH, DIM), jnp.float32),
                       pltpu.SemaphoreType.DMA,
                       pltpu.SemaphoreType.DMA,
                       pltpu.SemaphoreType.DMA,
                       pltpu.SemaphoreType.DMA],
    )(x, tok_sorted)

    hsh = pl.pallas_call(
        _sharedh_kernel,
        grid=(2 * HID // SCHUNK,),
        in_specs=[pl.BlockSpec((T, DIM), lambda c: (0, 0)),
                  pl.BlockSpec((SCHUNK, DIM), lambda c: (c, 0)),
                  pl.BlockSpec((SCHUNK, DIM), lambda c: (c, 0))],
        out_specs=pl.BlockSpec((T, SCHUNK), lambda c: (0, c)),
        out_shape=jax.ShapeDtypeStruct((T, 2 * HID), jnp.bfloat16),
        compiler_params=pltpu.CompilerParams(
            dimension_semantics=("arbitrary",)),
    )(xb16, sw1, sw2)

    ZT, ZK = 1024, 1024
    z = pl.pallas_call(
        _sharedz_kernel,
        grid=(T // ZT, 2 * HID // ZK),
        in_specs=[pl.BlockSpec((ZT, ZK), lambda t, k: (t, k)),
                  pl.BlockSpec((DIM, ZK), lambda t, k: (0, k))],
        out_specs=pl.BlockSpec((ZT, DIM), lambda t, k: (t, 0)),
        out_shape=jax.ShapeDtypeStruct((T, DIM), jnp.float32),
        compiler_params=pltpu.CompilerParams(
            dimension_semantics=("arbitrary", "arbitrary")),
    )(hsh, sw3)

    og = pl.pallas_call(
        _g_kernel,
        grid_spec=pltpu.PrefetchScalarGridSpec(
            num_scalar_prefetch=4,
            grid=(NTILES, HID // GCHUNK),
            in_specs=[
                pl.BlockSpec((TILE, DIM), lambda i, j, e, r, o, v: (r[i], 0)),
                pl.BlockSpec((TILE, 1), lambda i, j, e, r, o, v: (r[i], 0)),
                pl.BlockSpec((1, GCHUNK, DIM),
                             lambda i, j, e, r, o, v: (e[i], j, 0)),
                pl.BlockSpec((1, DIM, GCHUNK),
                             lambda i, j, e, r, o, v: (e[i], 0, j)),
                pl.BlockSpec((1, GCHUNK, DIM),
                             lambda i, j, e, r, o, v: (e[i], j, 0)),
            ],
            out_specs=pl.BlockSpec((TILE, DIM),
                                   lambda i, j, e, r, o, v: (o[i], 0)),
            scratch_shapes=[pltpu.VMEM((TILE, HID), jnp.float32)],
        ),
        out_shape=jax.ShapeDtypeStruct((P + TILE, DIM), jnp.float32),
        compiler_params=pltpu.CompilerParams(
            dimension_semantics=("arbitrary", "arbitrary")),
    )(eid, rin, rout, vld, xg, ws.reshape(P, 1), w1, w2, w3)

    y = pl.kernel(
        _combine_sc,
        out_type=jax.ShapeDtypeStruct((T, DIM), jnp.float32),
        mesh=mesh,
        scratch_types=[pltpu.VMEM((2 * T // NW,), jnp.int32),
                       pltpu.VMEM((2 * CB, DIM), jnp.float32),
                       pltpu.VMEM((2 * CB, DIM), jnp.float32),
                       pltpu.VMEM((CB, DIM), jnp.float32),
                       pltpu.VMEM((CB, DIM), jnp.float32),
                       pltpu.VMEM((CB, DIM), jnp.float32),
                       pltpu.VMEM((CB, DIM), jnp.float32),
                       pltpu.SemaphoreType.DMA,
                       pltpu.SemaphoreType.DMA,
                       pltpu.SemaphoreType.DMA,
                       pltpu.SemaphoreType.DMA,
                       pltpu.SemaphoreType.DMA,
                       pltpu.SemaphoreType.DMA],
    )(og, z, pos)
    return y


# speedup vs baseline: 1.5511x; 1.1404x over previous
import functools

import jax
import jax.numpy as jnp
from jax import lax
from jax.experimental import pallas as pl
from jax.experimental.pallas import tpu as pltpu
from jax.experimental.pallas import tpu_sc as plsc

NC = 2
NS = 16
NW = NC * NS
L = 16

DIM = 2048
HID = 2048
E = 16
TOPK = 2
T = 4096
NPAIR = T * TOPK
TILE = 256
NTILES = NPAIR // TILE + E
P = NTILES * TILE
GCHUNK = 1024
SCHUNK = 256


def _gate_kernel(x_ref, gw_ref, w_ref, i_ref, r_ref, off_ref, eid_ref,
                 carry_ref):
    step = pl.program_id(0)

    @pl.when(step == 0)
    def _():
        carry_ref[...] = jnp.zeros_like(carry_ref)

    @pl.when(step < T // TILE)
    def _():
        xb = x_ref[...]
        logits = jax.lax.dot_general(xb, gw_ref[...], (((1,), (1,)), ((), ())),
                                     preferred_element_type=jnp.float32)
        m = jnp.max(logits, axis=1, keepdims=True)
        p = jnp.exp(logits - m)
        s = p / jnp.sum(p, axis=1, keepdims=True)
        iota = jax.lax.broadcasted_iota(jnp.int32, s.shape, 1)
        m1 = jnp.max(s, axis=1, keepdims=True)
        i1 = jnp.min(jnp.where(s == m1, iota, E), axis=1, keepdims=True)
        s2 = jnp.where(iota == i1, -1.0, s)
        m2 = jnp.max(s2, axis=1, keepdims=True)
        i2 = jnp.min(jnp.where(s2 == m2, iota, E), axis=1, keepdims=True)
        w_ref[...] = jnp.concatenate([m1, m2], axis=1)
        i_ref[...] = jnp.concatenate([i1, i2], axis=1)
        oh0 = (i1 == iota).astype(jnp.float32)
        oh1 = (i2 == iota).astype(jnp.float32)
        ri = jax.lax.broadcasted_iota(jnp.int32, (TILE, TILE), 0)
        ci = jax.lax.broadcasted_iota(jnp.int32, (TILE, TILE), 1)
        tril = (ri > ci).astype(jnp.float32)
        carry = carry_ref[0:1, :]
        r0m = jax.lax.dot_general(tril, oh0, (((1,), (0,)), ((), ())),
                                  preferred_element_type=jnp.float32) + carry
        r0 = jnp.sum(r0m * oh0, axis=1, keepdims=True)
        carry = carry + jnp.sum(oh0, axis=0, keepdims=True)
        r1m = jax.lax.dot_general(tril, oh1, (((1,), (0,)), ((), ())),
                                  preferred_element_type=jnp.float32) + carry
        r1 = jnp.sum(r1m * oh1, axis=1, keepdims=True)
        carry = carry + jnp.sum(oh1, axis=0, keepdims=True)
        carry_ref[...] = jnp.broadcast_to(carry, carry_ref.shape)
        r_ref[...] = jnp.concatenate([r0, r1], axis=1).astype(jnp.int32)

    @pl.when(step == T // TILE)
    def _():
        counts = carry_ref[0:1, :].astype(jnp.int32)
        padded = ((counts + TILE - 1) // TILE) * TILE
        ri = jax.lax.broadcasted_iota(jnp.int32, (E, E), 0)
        ci = jax.lax.broadcasted_iota(jnp.int32, (E, E), 1)
        lm = (ri < ci).astype(jnp.float32)
        off = jax.lax.dot_general(padded.astype(jnp.float32), lm,
                                  (((1,), (0,)), ((), ())),
                                  preferred_element_type=jnp.float32)
        off = off.astype(jnp.int32)
        off_ref[...] = jnp.broadcast_to(off, off_ref.shape)
        ts = off // TILE
        eye = jnp.eye(E, dtype=jnp.float32)
        ts_sub = jax.lax.dot_general(eye, ts.astype(jnp.float32),
                                     (((1,), (1,)), ((), ())),
                                     preferred_element_type=jnp.float32)
        ti = jax.lax.broadcasted_iota(jnp.int32, (E, 64), 1).astype(
            jnp.float32)
        ge = (ti >= ts_sub).astype(jnp.float32)
        eid = jax.lax.dot_general(jnp.ones((1, E), jnp.float32), ge,
                                  (((1,), (0,)), ((), ())),
                                  preferred_element_type=jnp.float32) - 1.0
        eid = eid.astype(jnp.int32)
        iota_e = jax.lax.broadcasted_iota(jnp.int32, (1, E), 1)
        e_last = jnp.max(jnp.where(counts > 0, iota_e, 0), axis=1,
                         keepdims=True)
        eidm = jnp.minimum(eid, e_last)
        used = jnp.sum(padded, axis=1, keepdims=True) // TILE
        iota64 = jax.lax.broadcasted_iota(jnp.int32, (1, 64), 1)
        valid = (iota64 < used).astype(jnp.int32)
        rin = jnp.minimum(iota64, used - 1)
        rout = jnp.where(valid != 0, iota64, NTILES)
        meta = jnp.concatenate([eidm, rin, rout, valid], axis=1)
        eid_ref[...] = jnp.broadcast_to(meta, eid_ref.shape)


def _cast_kernel(x_ref, xb_ref):
    xb_ref[...] = x_ref[...].astype(jnp.bfloat16)


def _sharedh_kernel(x_ref, sw1_ref, sw2_ref, h_ref):
    xb = x_ref[...]
    s1b = sw1_ref[...].astype(jnp.bfloat16)
    s2b = sw2_ref[...].astype(jnp.bfloat16)
    a = jax.lax.dot_general(xb, s1b, (((1,), (1,)), ((), ())),
                            preferred_element_type=jnp.float32)
    b = jax.lax.dot_general(xb, s2b, (((1,), (1,)), ((), ())),
                            preferred_element_type=jnp.float32)
    h_ref[...] = (a * jax.nn.sigmoid(a) * b).astype(jnp.bfloat16)


def _sharedz_kernel(h_ref, sw3_ref, z_ref):
    k = pl.program_id(1)
    s3b = sw3_ref[...].astype(jnp.bfloat16)
    zc = jax.lax.dot_general(h_ref[...], s3b, (((1,), (1,)), ((), ())),
                             preferred_element_type=jnp.float32)

    @pl.when(k == 0)
    def _():
        z_ref[...] = zc

    @pl.when(k != 0)
    def _():
        z_ref[...] += zc


def _g1_kernel(eid_ref, rin_ref, rout_ref, vld_ref, xg_ref, ws_ref,
               w1_ref, w3_ref, h1_ref, h3_ref, w1b, w3b):
    del rin_ref, rout_ref
    i = pl.program_id(1)
    e = eid_ref[i]
    eprev = eid_ref[jnp.maximum(i, 1) - 1]

    @pl.when((i == 0) | (e != eprev))
    def _():
        w1b[...] = w1_ref[0].astype(jnp.bfloat16)
        w3b[...] = w3_ref[0].astype(jnp.bfloat16)

    @pl.when(vld_ref[i] != 0)
    def _():
        xi = (xg_ref[...] * ws_ref[...]).astype(jnp.bfloat16)
        a = jax.lax.dot_general(xi, w1b[...], (((1,), (1,)), ((), ())),
                                preferred_element_type=jnp.float32)
        h1_ref[...] = (a * jax.nn.sigmoid(a)).astype(jnp.bfloat16)
        h3_ref[...] = jax.lax.dot_general(
            xi, w3b[...], (((1,), (1,)), ((), ())),
            preferred_element_type=jnp.float32).astype(jnp.bfloat16)


def _g2_kernel(eid_ref, rin_ref, rout_ref, vld_ref, h1_ref, h3_ref, w2_ref,
               o_ref, w2b):
    del rin_ref, rout_ref
    i = pl.program_id(0)
    e = eid_ref[i]
    eprev = eid_ref[jnp.maximum(i, 1) - 1]

    @pl.when((i == 0) | (e != eprev))
    def _():
        w2b[...] = w2_ref[0].astype(jnp.bfloat16)

    @pl.when(vld_ref[i] != 0)
    def _():
        o_ref[...] = jax.lax.dot_general(
            h1_ref[...], w2b[...], (((1,), (1,)), ((), ())),
            preferred_element_type=jnp.float32) \
            * h3_ref[...].astype(jnp.float32)


def _worker_id():
    return lax.axis_index("s") * NC + lax.axis_index("c")


def _dispatch_sc(idx_hbm, wts_hbm, rank_hbm, off_hbm, tok_hbm, ws_hbm,
                 pos_hbm, idx_v, wts_v, rank_v, off_v, tok_v, ws_v, pos_v):
    @pl.when(_worker_id() == 0)
    def _():
        pltpu.sync_copy(idx_hbm, idx_v)
        pltpu.sync_copy(wts_hbm, wts_v)
        pltpu.sync_copy(rank_hbm, rank_v)
        pltpu.sync_copy(off_hbm.at[0], off_v)

        def zbody(i, _):
            sl = pl.ds(i * L, L)
            tok_v[sl] = jnp.zeros((L,), jnp.int32)
            ws_v[sl] = jnp.zeros((L,), jnp.float32)
            return ()
        lax.fori_loop(0, P // L, zbody, ())

        iota = lax.broadcasted_iota(jnp.int32, (L,), 0)

        def body(i, _):
            sl = pl.ds(i * L, L)
            e16 = idx_v[sl]
            o16 = plsc.load_gather(off_v, [e16])
            p16 = rank_v[sl] + o16
            pos_v[sl] = p16
            t16 = lax.shift_right_logical(iota + i * L, 1)
            plsc.store_scatter(tok_v, [p16], t16)
            plsc.store_scatter(ws_v, [p16], wts_v[sl])
            return ()
        lax.fori_loop(0, NPAIR // L, body, ())
        pltpu.sync_copy(tok_v, tok_hbm)
        pltpu.sync_copy(ws_v, ws_hbm)
        pltpu.sync_copy(pos_v, pos_hbm)


GCH = 24


def _gather_sc(x_hbm, tok_hbm, xg_hbm, idx_v, row_a, row_b,
               sga, sgb, ssa, ssb):
    rows_w = P // NW
    nch = rows_w // GCH
    base = _worker_id() * rows_w
    pltpu.sync_copy(tok_hbm.at[pl.ds(base, rows_w)], idx_v)

    def g_start(c, buf, sem):
        pltpu.make_async_copy(x_hbm.at[idx_v.at[pl.ds(c * GCH, GCH)]],
                              buf, sem).start()

    def s_start(c, buf, sem):
        pltpu.make_async_copy(buf, xg_hbm.at[pl.ds(base + c * GCH, GCH)],
                              sem).start()

    def g_wait(buf, sem):
        pltpu.make_async_copy(x_hbm.at[idx_v.at[pl.ds(0, GCH)]],
                              buf, sem).wait()

    def s_wait(buf, sem):
        pltpu.make_async_copy(buf, xg_hbm.at[pl.ds(base, GCH)], sem).wait()

    g_start(0, row_a, sga)

    def body(i, _):
        @pl.when(i > 0)
        def _():
            s_wait(row_b, ssb)
        g_start(2 * i + 1, row_b, sgb)
        g_wait(row_a, sga)
        s_start(2 * i, row_a, ssa)

        @pl.when(i < nch // 2 - 1)
        def _():
            s_wait(row_a, ssa)
            g_start(2 * i + 2, row_a, sga)
        g_wait(row_b, sgb)
        s_start(2 * i + 1, row_b, ssb)
        return ()
    lax.fori_loop(0, nch // 2, body, ())
    s_wait(row_a, ssa)
    s_wait(row_b, ssb)


CB = 4


def _combine_sc(og_hbm, z_hbm, pos_hbm, y_hbm, pos_v, row_a, row_b,
                z_a, z_b, y_a, y_b, sga, sgb, sza, szb, sya, syb):
    tok_w = T // NW
    nsb = tok_w // CB
    tb = _worker_id() * tok_w
    pltpu.sync_copy(pos_hbm.at[pl.ds(tb * 2, tok_w * 2)], pos_v)

    def fetch_start(s, rows, zv, sg, sz):
        t0 = tb + s * CB
        pltpu.make_async_copy(og_hbm.at[pos_v.at[pl.ds(s * 2 * CB, 2 * CB)]],
                              rows, sg).start()
        pltpu.make_async_copy(z_hbm.at[pl.ds(t0, CB)], zv, sz).start()

    def fetch_wait(rows, zv, sg, sz):
        pltpu.make_async_copy(og_hbm.at[pos_v.at[pl.ds(0, 2 * CB)]],
                              rows, sg).wait()
        pltpu.make_async_copy(z_hbm.at[pl.ds(tb, CB)], zv, sz).wait()

    def compute_store(s, rows, zv, yv, sy):
        def cbody(c, _):
            sl = pl.ds(c * L, L)
            for k in range(CB):
                yv[k, sl] = rows[2 * k, sl] + rows[2 * k + 1, sl] + zv[k, sl]
            return ()
        lax.fori_loop(0, DIM // L, cbody, ())
        pltpu.make_async_copy(yv, y_hbm.at[pl.ds(tb + s * CB, CB)],
                              sy).start()

    def y_wait(yv, sy):
        pltpu.make_async_copy(yv, y_hbm.at[pl.ds(tb, CB)], sy).wait()

    fetch_start(0, row_a, z_a, sga, sza)

    def body(i, _):
        @pl.when(i > 0)
        def _():
            y_wait(y_b, syb)
        fetch_start(2 * i + 1, row_b, z_b, sgb, szb)
        fetch_wait(row_a, z_a, sga, sza)
        compute_store(2 * i, row_a, z_a, y_a, sya)

        @pl.when(i < nsb // 2 - 1)
        def _():
            y_wait(y_a, sya)
            fetch_start(2 * i + 2, row_a, z_a, sga, sza)
        fetch_wait(row_b, z_b, sgb, szb)
        compute_store(2 * i + 1, row_b, z_b, y_b, syb)
        return ()
    lax.fori_loop(0, nsb // 2, body, ())
    y_wait(y_a, sya)
    y_wait(y_b, syb)


def kernel(x, gate_w, w1, b1, w2, b2, w3, b3, sw1, sw2, sw3):
    del b1, b2, b3
    nblk = T // TILE
    weights, indices, rank, off_m, eid_m = pl.pallas_call(
        _gate_kernel,
        grid=(nblk + 1,),
        in_specs=[pl.BlockSpec((TILE, DIM),
                               lambda i: (jnp.minimum(i, nblk - 1), 0)),
                  pl.BlockSpec((E, DIM), lambda i: (0, 0))],
        out_specs=[pl.BlockSpec((TILE, TOPK),
                                lambda i: (jnp.minimum(i, nblk - 1), 0)),
                   pl.BlockSpec((TILE, TOPK),
                                lambda i: (jnp.minimum(i, nblk - 1), 0)),
                   pl.BlockSpec((TILE, TOPK),
                                lambda i: (jnp.minimum(i, nblk - 1), 0)),
                   pl.BlockSpec((8, E), lambda i: (0, 0)),
                   pl.BlockSpec((8, 256), lambda i: (0, 0))],
        out_shape=[jax.ShapeDtypeStruct((T, TOPK), jnp.float32),
                   jax.ShapeDtypeStruct((T, TOPK), jnp.int32),
                   jax.ShapeDtypeStruct((T, TOPK), jnp.int32),
                   jax.ShapeDtypeStruct((8, E), jnp.int32),
                   jax.ShapeDtypeStruct((8, 256), jnp.int32)],
        scratch_shapes=[pltpu.VMEM((8, E), jnp.float32)],
    )(x, gate_w)

    eid = eid_m[0, :NTILES]
    rin = eid_m[0, 64:64 + NTILES]
    rout = eid_m[0, 128:128 + NTILES]
    vld = eid_m[0, 192:192 + NTILES]
    mesh = plsc.VectorSubcoreMesh(core_axis_name="c", subcore_axis_name="s",
                                  num_cores=NC, num_subcores=NS)
    tok_sorted, ws, pos = pl.kernel(
        _dispatch_sc,
        out_type=[jax.ShapeDtypeStruct((P,), jnp.int32),
                  jax.ShapeDtypeStruct((P,), jnp.float32),
                  jax.ShapeDtypeStruct((NPAIR,), jnp.int32)],
        mesh=mesh,
        scratch_types=[pltpu.VMEM((NPAIR,), jnp.int32),
                       pltpu.VMEM((NPAIR,), jnp.float32),
                       pltpu.VMEM((NPAIR,), jnp.int32),
                       pltpu.VMEM((E,), jnp.int32),
                       pltpu.VMEM((P,), jnp.int32),
                       pltpu.VMEM((P,), jnp.float32),
                       pltpu.VMEM((NPAIR,), jnp.int32)],
        compiler_params=pltpu.CompilerParams(needs_layout_passes=False),
    )(indices.reshape(-1), weights.reshape(-1), rank.reshape(-1), off_m)

    xb16 = pl.pallas_call(
        _cast_kernel,
        grid=(T // TILE,),
        in_specs=[pl.BlockSpec((TILE, DIM), lambda i: (i, 0))],
        out_specs=pl.BlockSpec((TILE, DIM), lambda i: (i, 0)),
        out_shape=jax.ShapeDtypeStruct((T, DIM), jnp.bfloat16),
    )(x)

    xg = pl.kernel(
        _gather_sc,
        out_type=jax.ShapeDtypeStruct((P, DIM), jnp.float32),
        mesh=mesh,
        scratch_types=[pltpu.VMEM((P // NW,), jnp.int32),
                       pltpu.VMEM((GCH, DIM), jnp.float32),
                       pltpu.VMEM((GCH, DIM), jnp.float32),
                       pltpu.SemaphoreType.DMA,
                       pltpu.SemaphoreType.DMA,
                       pltpu.SemaphoreType.DMA,
                       pltpu.SemaphoreType.DMA],
    )(x, tok_sorted)

    hsh = pl.pallas_call(
        _sharedh_kernel,
        grid=(2 * HID // SCHUNK,),
        in_specs=[pl.BlockSpec((T, DIM), lambda c: (0, 0)),
                  pl.BlockSpec((SCHUNK, DIM), lambda c: (c, 0)),
                  pl.BlockSpec((SCHUNK, DIM), lambda c: (c, 0))],
        out_specs=pl.BlockSpec((T, SCHUNK), lambda c: (0, c)),
        out_shape=jax.ShapeDtypeStruct((T, 2 * HID), jnp.bfloat16),
        compiler_params=pltpu.CompilerParams(
            dimension_semantics=("arbitrary",)),
    )(xb16, sw1, sw2)

    ZT, ZK = 1024, 1024
    z = pl.pallas_call(
        _sharedz_kernel,
        grid=(T // ZT, 2 * HID // ZK),
        in_specs=[pl.BlockSpec((ZT, ZK), lambda t, k: (t, k)),
                  pl.BlockSpec((DIM, ZK), lambda t, k: (0, k))],
        out_specs=pl.BlockSpec((ZT, DIM), lambda t, k: (t, 0)),
        out_shape=jax.ShapeDtypeStruct((T, DIM), jnp.float32),
        compiler_params=pltpu.CompilerParams(
            dimension_semantics=("arbitrary", "arbitrary")),
    )(hsh, sw3)

    h1, h3 = pl.pallas_call(
        _g1_kernel,
        grid_spec=pltpu.PrefetchScalarGridSpec(
            num_scalar_prefetch=4,
            grid=(HID // GCHUNK, NTILES),
            in_specs=[
                pl.BlockSpec((TILE, DIM), lambda j, i, e, r, o, v: (r[i], 0)),
                pl.BlockSpec((TILE, 1), lambda j, i, e, r, o, v: (r[i], 0)),
                pl.BlockSpec((1, GCHUNK, DIM),
                             lambda j, i, e, r, o, v: (e[i], j, 0)),
                pl.BlockSpec((1, GCHUNK, DIM),
                             lambda j, i, e, r, o, v: (e[i], j, 0)),
            ],
            out_specs=[
                pl.BlockSpec((TILE, GCHUNK),
                             lambda j, i, e, r, o, v: (o[i], j)),
                pl.BlockSpec((TILE, GCHUNK),
                             lambda j, i, e, r, o, v: (o[i], j)),
            ],
            scratch_shapes=[pltpu.VMEM((GCHUNK, DIM), jnp.bfloat16),
                            pltpu.VMEM((GCHUNK, DIM), jnp.bfloat16)],
        ),
        out_shape=[jax.ShapeDtypeStruct((P + TILE, HID), jnp.bfloat16),
                   jax.ShapeDtypeStruct((P + TILE, HID), jnp.bfloat16)],
        compiler_params=pltpu.CompilerParams(
            dimension_semantics=("arbitrary", "arbitrary")),
    )(eid, rin, rout, vld, xg, ws.reshape(P, 1), w1, w3)

    og = pl.pallas_call(
        _g2_kernel,
        grid_spec=pltpu.PrefetchScalarGridSpec(
            num_scalar_prefetch=4,
            grid=(NTILES,),
            in_specs=[
                pl.BlockSpec((TILE, HID), lambda i, e, r, o, v: (r[i], 0)),
                pl.BlockSpec((TILE, HID), lambda i, e, r, o, v: (r[i], 0)),
                pl.BlockSpec((1, DIM, HID),
                             lambda i, e, r, o, v: (e[i], 0, 0)),
            ],
            out_specs=pl.BlockSpec((TILE, DIM),
                                   lambda i, e, r, o, v: (o[i], 0)),
            scratch_shapes=[pltpu.VMEM((DIM, HID), jnp.bfloat16)],
        ),
        out_shape=jax.ShapeDtypeStruct((P + TILE, DIM), jnp.float32),
        compiler_params=pltpu.CompilerParams(
            dimension_semantics=("arbitrary",)),
    )(eid, rin, rout, vld, h1, h3, w2)

    y = pl.kernel(
        _combine_sc,
        out_type=jax.ShapeDtypeStruct((T, DIM), jnp.float32),
        mesh=mesh,
        scratch_types=[pltpu.VMEM((2 * T // NW,), jnp.int32),
                       pltpu.VMEM((2 * CB, DIM), jnp.float32),
                       pltpu.VMEM((2 * CB, DIM), jnp.float32),
                       pltpu.VMEM((CB, DIM), jnp.float32),
                       pltpu.VMEM((CB, DIM), jnp.float32),
                       pltpu.VMEM((CB, DIM), jnp.float32),
                       pltpu.VMEM((CB, DIM), jnp.float32),
                       pltpu.SemaphoreType.DMA,
                       pltpu.SemaphoreType.DMA,
                       pltpu.SemaphoreType.DMA,
                       pltpu.SemaphoreType.DMA,
                       pltpu.SemaphoreType.DMA,
                       pltpu.SemaphoreType.DMA],
    )(og, z, pos)
    return y
